# baseline (speedup 1.0000x reference)
import functools

import jax
import jax.numpy as jnp
from jax import lax
from jax.experimental import pallas as pl
from jax.experimental.pallas import tpu as pltpu
from jax.experimental.pallas import tpu_sc as plsc

B = 16384
D = 16

_info = plsc.get_sparse_core_info()
_NC, _NS = _info.num_cores, _info.num_subcores
NW = _NC * _NS
BPW = B // NW
CHUNK = 128
NCHUNK = BPW // CHUNK

_sc_mesh = plsc.VectorSubcoreMesh(core_axis_name="c", subcore_axis_name="s")


@functools.partial(
    pl.kernel,
    mesh=_sc_mesh,
    out_type=[
        jax.ShapeDtypeStruct((D, B), jnp.float32),
        jax.ShapeDtypeStruct((D, B), jnp.float32),
    ],
    scratch_types=[
        pltpu.VMEM((NCHUNK, CHUNK), jnp.int32),
        pltpu.VMEM((D, BPW), jnp.float32),
        pltpu.VMEM((NCHUNK, CHUNK), jnp.int32),
        pltpu.VMEM((D, BPW), jnp.float32),
        pltpu.SemaphoreType.DMA,
        pltpu.SemaphoreType.DMA,
    ],
    compiler_params=pltpu.CompilerParams(use_tc_tiling_on_sc=False),
)
def _sc_gather(uid_hbm, mid_hbm, uid_tab, mid_tab, euid_out, emid_out,
               uidx_v, urows_v, midx_v, mrows_v, usem, msem):
    wid = lax.axis_index("s") * _NC + lax.axis_index("c")
    base = wid * BPW
    pltpu.sync_copy(uid_hbm.at[wid], uidx_v)
    pltpu.sync_copy(mid_hbm.at[wid], midx_v)
    copies = []
    for f in range(D):
        for ci in range(NCHUNK):
            copies.append(pltpu.async_copy(
                uid_tab.at[f].at[uidx_v.at[ci]],
                urows_v.at[f, pl.ds(ci * CHUNK, CHUNK)], usem))
            copies.append(pltpu.async_copy(
                mid_tab.at[f].at[midx_v.at[ci]],
                mrows_v.at[f, pl.ds(ci * CHUNK, CHUNK)], msem))
    for c in copies:
        c.wait()
    pltpu.sync_copy(urows_v, euid_out.at[:, pl.ds(base, BPW)])
    pltpu.sync_copy(mrows_v, emid_out.at[:, pl.ds(base, BPW)])


BLK = 2048
GRID = B // BLK


def _tc_body(gidx_ref, aidx_ref, jidx_ref, euid_ref, emid_ref,
             gt_ref, at_ref, jt_ref, w1_ref, b1_ref, w2_ref,
             wsvd_ref, bsvd_ref, out_ref, l1_ref, l2_ref, l3_ref):
    euid = euid_ref[...]
    emid = emid_ref[...]
    g = gidx_ref[0, 0, :]
    a = aidx_ref[0, 0, :]
    j = jidx_ref[0, 0, :]
    tab = jnp.concatenate([gt_ref[...], at_ref[...], jt_ref[...]], axis=0)
    w1 = w1_ref[...]
    w1a = w1[:D, :]
    w1b = w1[D:, :]
    b1col = b1_ref[...][:, None]
    w2col = w2_ref[...]
    wsvd = wsvd_ref[...]
    wsvda = wsvd[:D, :]
    wsvdb = wsvd[D:, :]
    bsvd = bsvd_ref[0]

    def ddt(lhs, rhs):
        return lax.dot_general(lhs, rhs, (((0,), (0,)), ((), ())),
                               preferred_element_type=jnp.float32)

    iota = lax.broadcasted_iota(jnp.int32, (30, BLK), 0)
    oh_g = (g[None, :] == iota).astype(jnp.float32)
    oh_a = ((a[None, :] + 2) == iota).astype(jnp.float32)
    oh_j = ((j[None, :] + 9) == iota).astype(jnp.float32)
    eg = ddt(tab, oh_g)
    ea = ddt(tab, oh_a)
    ej = ddt(tab, oh_j)

    m1 = ddt(w1a, emid) + b1col

    def score(e):
        h = jnp.tanh(m1 + ddt(w1b, e))
        return jnp.sum(h * w2col, axis=0, keepdims=True)

    s1, s2, s3 = score(eg), score(ea), score(ej)
    mx = jnp.maximum(jnp.maximum(s1, s2), s3)
    x1 = jnp.exp(s1 - mx)
    x2 = jnp.exp(s2 - mx)
    x3 = jnp.exp(s3 - mx)
    den = x1 + x2 + x3
    l1, l2, l3 = x1 / den, x2 / den, x3 / den

    fu = l1 * eg + l2 * ea + l3 * ej + euid
    out_ref[...] = (jnp.sum(fu * wsvda + emid * wsvdb, axis=0, keepdims=True)
                    + bsvd)
    l1_ref[...] = l1
    l2_ref[...] = l2
    l3_ref[...] = l3


def kernel(uid_table, gender_table, age_table, job_table, mid_table,
           W1, b1, W2, b2, W_svd, b_svd,
           uid, gender, age, job, mid):
    uid = uid.astype(jnp.int32).reshape(NW, NCHUNK, CHUNK)
    mid = mid.astype(jnp.int32).reshape(NW, NCHUNK, CHUNK)
    euidT, emidT = _sc_gather(uid, mid, uid_table.T, mid_table.T)

    g3 = gender.astype(jnp.int32).reshape(GRID, 1, BLK)
    a3 = age.astype(jnp.int32).reshape(GRID, 1, BLK)
    j3 = job.astype(jnp.int32).reshape(GRID, 1, BLK)

    idx_spec = pl.BlockSpec((1, 1, BLK), lambda i: (i, 0, 0))
    colblk = pl.BlockSpec((D, BLK), lambda i: (0, i))
    full = lambda shape: pl.BlockSpec(shape, lambda i: tuple(0 for _ in shape))
    row1 = pl.BlockSpec((1, BLK), lambda i: (0, i))

    out, l1, l2, l3 = pl.pallas_call(
        _tc_body,
        grid=(GRID,),
        in_specs=[idx_spec, idx_spec, idx_spec, colblk, colblk,
                  full((2, D)), full((7, D)), full((21, D)),
                  full((2 * D, D)), full((D,)), full((D, 1)),
                  full((2 * D, 1)), full((1,))],
        out_specs=[row1, row1, row1, row1],
        out_shape=[jax.ShapeDtypeStruct((1, B), jnp.float32),
                   jax.ShapeDtypeStruct((1, B), jnp.float32),
                   jax.ShapeDtypeStruct((1, B), jnp.float32),
                   jax.ShapeDtypeStruct((1, B), jnp.float32)],
    )(g3, a3, j3, euidT, emidT,
      gender_table, age_table, job_table, W1, b1, W2, W_svd, b_svd)

    lam = jnp.stack([l1[0], l2[0], l3[0]], axis=1).reshape(B, 3, 1)
    return (out.reshape(B, 1), lam)


# speedup vs baseline: 1.4730x; 1.4730x over previous
import functools

import jax
import jax.numpy as jnp
from jax import lax
from jax.experimental import pallas as pl
from jax.experimental.pallas import tpu as pltpu
from jax.experimental.pallas import tpu_sc as plsc

B = 16384
D = 16
V = 1000000
PR = V // 8

_info = plsc.get_sparse_core_info()
_NC, _NS, _L = _info.num_cores, _info.num_subcores, _info.num_lanes
NW = _NC * _NS
BPW = B // NW
CH = 256
NCHT = BPW // CH


RW = 8192
RGRID = -(-V // RW)


def _repack_body(in_ref, out_ref):
    x = in_ref[...]
    y = x.reshape(D, RW // 8, 8)
    out_ref[...] = jnp.transpose(y, (1, 2, 0)).reshape(RW // 8, 8 * D)


def _repack(tabT):
    return pl.pallas_call(
        _repack_body,
        grid=(RGRID,),
        in_specs=[pl.BlockSpec((D, RW), lambda i: (0, i))],
        out_specs=pl.BlockSpec((RW // 8, 8 * D), lambda i: (i, 0)),
        out_shape=jax.ShapeDtypeStruct((PR, 8 * D), jnp.float32),
    )(tabT)


_sc_mesh = plsc.VectorSubcoreMesh(core_axis_name="c", subcore_axis_name="s")


@functools.partial(
    pl.kernel,
    mesh=_sc_mesh,
    out_type=[
        jax.ShapeDtypeStruct((NW, D * BPW), jnp.float32),
        jax.ShapeDtypeStruct((NW, D * BPW), jnp.float32),
    ],
    scratch_types=[
        pltpu.VMEM((BPW,), jnp.int32),
        pltpu.VMEM((BPW,), jnp.int32),
        pltpu.VMEM((BPW,), jnp.int32),
        pltpu.VMEM((BPW,), jnp.int32),
        pltpu.VMEM((BPW,), jnp.int32),
        pltpu.VMEM((BPW,), jnp.int32),
        pltpu.VMEM((2, CH, 8 * D), jnp.float32),
        pltpu.VMEM((D * BPW,), jnp.float32),
        pltpu.VMEM((D * BPW,), jnp.float32),
        pltpu.SemaphoreType.DMA,
    ],
    compiler_params=pltpu.CompilerParams(use_tc_tiling_on_sc=False,
                                         needs_layout_passes=False),
)
def _sc_gather(uid_hbm, mid_hbm, up_tab, mp_tab, euid_out, emid_out,
               uidx_v, midx_v, urow_v, mrow_v, ulb_v, mlb_v,
               buf, urows_v, mrows_v, sem):
    wid = lax.axis_index("s") * _NC + lax.axis_index("c")
    pltpu.sync_copy(uid_hbm.at[wid], uidx_v)
    pltpu.sync_copy(mid_hbm.at[wid], midx_v)

    for idx_v, row_v, lb_v in ((uidx_v, urow_v, ulb_v),
                               (midx_v, mrow_v, mlb_v)):
        for v in range(BPW // _L):
            raw = idx_v[pl.ds(v * _L, _L)]
            row_v[pl.ds(v * _L, _L)] = jnp.right_shift(raw, 3)
            lb_v[pl.ds(v * _L, _L)] = jnp.left_shift(
                jnp.bitwise_and(raw, 7), 4)

    def fire(tab, row_v, c, slot):
        return pltpu.async_copy(
            tab.at[row_v.at[pl.ds(c * CH, CH)]], buf.at[slot], sem)

    def extract(rows_v, lb_v, c, slot):
        def body(v, carry):
            rows = lax.broadcasted_iota(jnp.int32, (_L,), 0) + v * _L
            lb = lb_v[pl.ds(c * CH + v * _L, _L)]
            for f in range(D):
                vals = plsc.load_gather(buf.at[slot], [rows, lb + f])
                rows_v[pl.ds(f * BPW + c * CH + v * _L, _L)] = vals
            return carry

        lax.fori_loop(0, CH // _L, body, 0)

    cu0 = fire(up_tab, urow_v, 0, 0)
    cu1 = fire(up_tab, urow_v, 1, 1)
    cu0.wait()
    extract(urows_v, ulb_v, 0, 0)
    cm0 = fire(mp_tab, mrow_v, 0, 0)
    cu1.wait()
    extract(urows_v, ulb_v, 1, 1)
    cm1 = fire(mp_tab, mrow_v, 1, 1)
    cm0.wait()
    extract(mrows_v, mlb_v, 0, 0)
    cm1.wait()
    extract(mrows_v, mlb_v, 1, 1)

    pltpu.sync_copy(urows_v, euid_out.at[wid])
    pltpu.sync_copy(mrows_v, emid_out.at[wid])


BLK = BPW
GRID = B // BLK


def _tc_body(gidx_ref, aidx_ref, jidx_ref, euid_ref, emid_ref,
             gt_ref, at_ref, jt_ref, w1_ref, b1_ref, w2_ref,
             wsvd_ref, bsvd_ref, out_ref, l1_ref, l2_ref, l3_ref):
    euid = euid_ref[0]
    emid = emid_ref[0]
    g = gidx_ref[0, 0, :]
    a = aidx_ref[0, 0, :]
    j = jidx_ref[0, 0, :]
    tab = jnp.concatenate([gt_ref[...], at_ref[...], jt_ref[...]], axis=0)
    w1 = w1_ref[...]
    w1a = w1[:D, :]
    w1b = w1[D:, :]
    b1col = b1_ref[...][:, None]
    w2col = w2_ref[...]
    wsvd = wsvd_ref[...]
    wsvda = wsvd[:D, :]
    wsvdb = wsvd[D:, :]
    bsvd = bsvd_ref[0]

    def ddt(lhs, rhs):
        return lax.dot_general(lhs, rhs, (((0,), (0,)), ((), ())),
                               preferred_element_type=jnp.float32)

    iota = lax.broadcasted_iota(jnp.int32, (30, BLK), 0)
    oh_g = (g[None, :] == iota).astype(jnp.float32)
    oh_a = ((a[None, :] + 2) == iota).astype(jnp.float32)
    oh_j = ((j[None, :] + 9) == iota).astype(jnp.float32)
    eg = ddt(tab, oh_g)
    ea = ddt(tab, oh_a)
    ej = ddt(tab, oh_j)

    m1 = ddt(w1a, emid) + b1col

    def score(e):
        h = jnp.tanh(m1 + ddt(w1b, e))
        return jnp.sum(h * w2col, axis=0, keepdims=True)

    s1, s2, s3 = score(eg), score(ea), score(ej)
    mx = jnp.maximum(jnp.maximum(s1, s2), s3)
    x1 = jnp.exp(s1 - mx)
    x2 = jnp.exp(s2 - mx)
    x3 = jnp.exp(s3 - mx)
    den = x1 + x2 + x3
    l1, l2, l3 = x1 / den, x2 / den, x3 / den

    fu = l1 * eg + l2 * ea + l3 * ej + euid
    out_ref[...] = (jnp.sum(fu * wsvda + emid * wsvdb, axis=0, keepdims=True)
                    + bsvd)
    l1_ref[...] = l1
    l2_ref[...] = l2
    l3_ref[...] = l3


def kernel(uid_table, gender_table, age_table, job_table, mid_table,
           W1, b1, W2, b2, W_svd, b_svd,
           uid, gender, age, job, mid):
    up = _repack(uid_table.T)
    mp = _repack(mid_table.T)

    uid = uid.astype(jnp.int32).reshape(NW, BPW)
    mid = mid.astype(jnp.int32).reshape(NW, BPW)
    euid2, emid2 = _sc_gather(uid, mid, up, mp)
    euid3 = euid2.reshape(NW, D, BPW)
    emid3 = emid2.reshape(NW, D, BPW)

    g3 = gender.astype(jnp.int32).reshape(GRID, 1, BLK)
    a3 = age.astype(jnp.int32).reshape(GRID, 1, BLK)
    j3 = job.astype(jnp.int32).reshape(GRID, 1, BLK)

    idx_spec = pl.BlockSpec((1, 1, BLK), lambda i: (i, 0, 0))
    colblk = pl.BlockSpec((1, D, BLK), lambda i: (i, 0, 0))
    full = lambda shape: pl.BlockSpec(shape, lambda i: tuple(0 for _ in shape))
    row1 = pl.BlockSpec((1, BLK), lambda i: (0, i))

    out, l1, l2, l3 = pl.pallas_call(
        _tc_body,
        grid=(GRID,),
        in_specs=[idx_spec, idx_spec, idx_spec, colblk, colblk,
                  full((2, D)), full((7, D)), full((21, D)),
                  full((2 * D, D)), full((D,)), full((D, 1)),
                  full((2 * D, 1)), full((1,))],
        out_specs=[row1, row1, row1, row1],
        out_shape=[jax.ShapeDtypeStruct((1, B), jnp.float32),
                   jax.ShapeDtypeStruct((1, B), jnp.float32),
                   jax.ShapeDtypeStruct((1, B), jnp.float32),
                   jax.ShapeDtypeStruct((1, B), jnp.float32)],
    )(g3, a3, j3, euid3, emid3,
      gender_table, age_table, job_table, W1, b1, W2, W_svd, b_svd)

    lam = jnp.stack([l1[0], l2[0], l3[0]], axis=1).reshape(B, 3, 1)
    return (out.reshape(B, 1), lam)


# speedup vs baseline: 3.0578x; 2.0759x over previous
import functools

import jax
import jax.numpy as jnp
from jax import lax
from jax.experimental import pallas as pl
from jax.experimental.pallas import tpu as pltpu
from jax.experimental.pallas import tpu_sc as plsc

B = 16384
D = 16
V = 1000000
PR = V // 8

_info = plsc.get_sparse_core_info()
_NC, _NS, _L = _info.num_cores, _info.num_subcores, _info.num_lanes
NW = _NC * _NS
BPW = B // NW
CH = 256
NCHT = BPW // CH


RW = 8192
RGRID = -(-V // RW)


_sc_mesh = plsc.VectorSubcoreMesh(core_axis_name="c", subcore_axis_name="s")


@functools.partial(
    pl.kernel,
    mesh=_sc_mesh,
    out_type=[
        jax.ShapeDtypeStruct((NW, D * BPW), jnp.float32),
        jax.ShapeDtypeStruct((NW, D * BPW), jnp.float32),
    ],
    scratch_types=[
        pltpu.VMEM((BPW,), jnp.int32),
        pltpu.VMEM((BPW,), jnp.int32),
        pltpu.VMEM((BPW,), jnp.int32),
        pltpu.VMEM((BPW,), jnp.int32),
        pltpu.VMEM((BPW,), jnp.int32),
        pltpu.VMEM((BPW,), jnp.int32),
        pltpu.VMEM((2, CH, 8 * D), jnp.float32),
        pltpu.VMEM((D * BPW,), jnp.float32),
        pltpu.VMEM((D * BPW,), jnp.float32),
        pltpu.SemaphoreType.DMA,
    ],
    compiler_params=pltpu.CompilerParams(use_tc_tiling_on_sc=False,
                                         needs_layout_passes=False),
)
def _sc_gather(uid_hbm, mid_hbm, up_tab, mp_tab, euid_out, emid_out,
               uidx_v, midx_v, urow_v, mrow_v, ulb_v, mlb_v,
               buf, urows_v, mrows_v, sem):
    wid = lax.axis_index("s") * _NC + lax.axis_index("c")
    pltpu.sync_copy(uid_hbm.at[wid], uidx_v)
    pltpu.sync_copy(mid_hbm.at[wid], midx_v)

    for idx_v, row_v, lb_v in ((uidx_v, urow_v, ulb_v),
                               (midx_v, mrow_v, mlb_v)):
        for v in range(BPW // _L):
            raw = idx_v[pl.ds(v * _L, _L)]
            row_v[pl.ds(v * _L, _L)] = jnp.right_shift(raw, 3)
            lb_v[pl.ds(v * _L, _L)] = jnp.left_shift(
                jnp.bitwise_and(raw, 7), 4)

    def fire(tab, row_v, c, slot):
        return pltpu.async_copy(
            tab.at[row_v.at[pl.ds(c * CH, CH)]], buf.at[slot], sem)

    def extract(rows_v, lb_v, c, slot):
        def body(v, carry):
            rows = lax.broadcasted_iota(jnp.int32, (_L,), 0) + v * _L
            lb = lb_v[pl.ds(c * CH + v * _L, _L)]
            for f in range(D):
                vals = plsc.load_gather(buf.at[slot], [rows, lb + f])
                rows_v[pl.ds(f * BPW + c * CH + v * _L, _L)] = vals
            return carry

        lax.fori_loop(0, CH // _L, body, 0)

    cu0 = fire(up_tab, urow_v, 0, 0)
    cu1 = fire(up_tab, urow_v, 1, 1)
    cu0.wait()
    extract(urows_v, ulb_v, 0, 0)
    cm0 = fire(mp_tab, mrow_v, 0, 0)
    cu1.wait()
    extract(urows_v, ulb_v, 1, 1)
    cm1 = fire(mp_tab, mrow_v, 1, 1)
    cm0.wait()
    extract(mrows_v, mlb_v, 0, 0)
    cm1.wait()
    extract(mrows_v, mlb_v, 1, 1)

    pltpu.sync_copy(urows_v, euid_out.at[wid])
    pltpu.sync_copy(mrows_v, emid_out.at[wid])


BLK = BPW
GRID = B // BLK


def _tc_body(gidx_ref, aidx_ref, jidx_ref, euid_ref, emid_ref,
             gt_ref, at_ref, jt_ref, w1_ref, b1_ref, w2_ref,
             wsvd_ref, bsvd_ref, out_ref, l1_ref, l2_ref, l3_ref):
    euid = euid_ref[0]
    emid = emid_ref[0]
    g = gidx_ref[0, 0, :]
    a = aidx_ref[0, 0, :]
    j = jidx_ref[0, 0, :]
    tab = jnp.concatenate([gt_ref[...], at_ref[...], jt_ref[...]], axis=0)
    w1 = w1_ref[...]
    w1a = w1[:D, :]
    w1b = w1[D:, :]
    b1col = b1_ref[...][:, None]
    w2col = w2_ref[...]
    wsvd = wsvd_ref[...]
    wsvda = wsvd[:D, :]
    wsvdb = wsvd[D:, :]
    bsvd = bsvd_ref[0]

    def ddt(lhs, rhs):
        return lax.dot_general(lhs, rhs, (((0,), (0,)), ((), ())),
                               preferred_element_type=jnp.float32)

    iota = lax.broadcasted_iota(jnp.int32, (30, BLK), 0)
    oh_g = (g[None, :] == iota).astype(jnp.float32)
    oh_a = ((a[None, :] + 2) == iota).astype(jnp.float32)
    oh_j = ((j[None, :] + 9) == iota).astype(jnp.float32)
    eg = ddt(tab, oh_g)
    ea = ddt(tab, oh_a)
    ej = ddt(tab, oh_j)

    m1 = ddt(w1a, emid) + b1col

    def score(e):
        h = jnp.tanh(m1 + ddt(w1b, e))
        return jnp.sum(h * w2col, axis=0, keepdims=True)

    s1, s2, s3 = score(eg), score(ea), score(ej)
    mx = jnp.maximum(jnp.maximum(s1, s2), s3)
    x1 = jnp.exp(s1 - mx)
    x2 = jnp.exp(s2 - mx)
    x3 = jnp.exp(s3 - mx)
    den = x1 + x2 + x3
    l1, l2, l3 = x1 / den, x2 / den, x3 / den

    fu = l1 * eg + l2 * ea + l3 * ej + euid
    out_ref[...] = (jnp.sum(fu * wsvda + emid * wsvdb, axis=0, keepdims=True)
                    + bsvd)
    l1_ref[...] = l1
    l2_ref[...] = l2
    l3_ref[...] = l3


def kernel(uid_table, gender_table, age_table, job_table, mid_table,
           W1, b1, W2, b2, W_svd, b_svd,
           uid, gender, age, job, mid):
    up = uid_table.reshape(PR, 8 * D)
    mp = mid_table.reshape(PR, 8 * D)

    uid = uid.astype(jnp.int32).reshape(NW, BPW)
    mid = mid.astype(jnp.int32).reshape(NW, BPW)
    euid2, emid2 = _sc_gather(uid, mid, up, mp)
    euid3 = euid2.reshape(NW, D, BPW)
    emid3 = emid2.reshape(NW, D, BPW)

    g3 = gender.astype(jnp.int32).reshape(GRID, 1, BLK)
    a3 = age.astype(jnp.int32).reshape(GRID, 1, BLK)
    j3 = job.astype(jnp.int32).reshape(GRID, 1, BLK)

    idx_spec = pl.BlockSpec((1, 1, BLK), lambda i: (i, 0, 0))
    colblk = pl.BlockSpec((1, D, BLK), lambda i: (i, 0, 0))
    full = lambda shape: pl.BlockSpec(shape, lambda i: tuple(0 for _ in shape))
    row1 = pl.BlockSpec((1, BLK), lambda i: (0, i))

    out, l1, l2, l3 = pl.pallas_call(
        _tc_body,
        grid=(GRID,),
        in_specs=[idx_spec, idx_spec, idx_spec, colblk, colblk,
                  full((2, D)), full((7, D)), full((21, D)),
                  full((2 * D, D)), full((D,)), full((D, 1)),
                  full((2 * D, 1)), full((1,))],
        out_specs=[row1, row1, row1, row1],
        out_shape=[jax.ShapeDtypeStruct((1, B), jnp.float32),
                   jax.ShapeDtypeStruct((1, B), jnp.float32),
                   jax.ShapeDtypeStruct((1, B), jnp.float32),
                   jax.ShapeDtypeStruct((1, B), jnp.float32)],
    )(g3, a3, j3, euid3, emid3,
      gender_table, age_table, job_table, W1, b1, W2, W_svd, b_svd)

    lam = jnp.stack([l1[0], l2[0], l3[0]], axis=1).reshape(B, 3, 1)
    return (out.reshape(B, 1), lam)
